# baseline (speedup 1.0000x reference)
import functools

import jax
import jax.numpy as jnp
from jax import lax
from jax.experimental import pallas as pl
from jax.experimental.pallas import tpu as pltpu
from jax.experimental.pallas import tpu_sc as plsc

VOCAB = 100000
SEQ = 200
DIM = 64
DIMP = 128
BATCH = 1024
LANES = 16

NUM_CORES = 2
NUM_SUBCORES = 16
NW = NUM_CORES * NUM_SUBCORES
TOK_PER_W = BATCH * SEQ // NW
CHUNK = SEQ
NCHUNK = TOK_PER_W // CHUNK
PAIRS = CHUNK // 2
GROUPS = DIM // LANES

_mesh = plsc.VectorSubcoreMesh(
    core_axis_name="c", subcore_axis_name="s",
    num_cores=NUM_CORES, num_subcores=NUM_SUBCORES)


@functools.partial(
    pl.kernel,
    out_type=jax.ShapeDtypeStruct((BATCH * SEQ * DIM,), jnp.float32),
    mesh=_mesh,
    scratch_types=[
        pltpu.VMEM((TOK_PER_W,), jnp.int32),
        [pltpu.VMEM((CHUNK, DIMP), jnp.float32) for _ in range(2)],
        [pltpu.VMEM((CHUNK * DIM,), jnp.float32) for _ in range(2)],
        [pltpu.SemaphoreType.DMA for _ in range(2)],
        [pltpu.SemaphoreType.DMA for _ in range(2)],
        [pltpu.SemaphoreType.DMA for _ in range(2)],
    ],
)
def _emb_kernel(idx_hbm, table_hbm, pos_hbm, out_hbm,
                idx_v, tok_bufs, out_bufs, gsems, psems, osems):
    wid = lax.axis_index("s") * NUM_CORES + lax.axis_index("c")
    base = wid * TOK_PER_W

    pltpu.sync_copy(idx_hbm.at[pl.ds(base, TOK_PER_W)], idx_v)

    def start_chunk(k, b):
        g = pltpu.async_copy(
            table_hbm.at[idx_v.at[pl.ds(k * CHUNK, CHUNK)]], tok_bufs[b],
            gsems[b])
        p = pltpu.async_copy(pos_hbm, out_bufs[b], psems[b])
        return g, p

    inflight = {0: start_chunk(0, 0)}
    out_dma = {}
    for k in range(NCHUNK):
        cur = k % 2
        if k + 1 < NCHUNK:
            if k >= 1:
                out_dma.pop(k - 1).wait()
            inflight[k + 1] = start_chunk(k + 1, 1 - cur)
        g, p = inflight.pop(k)
        g.wait()
        p.wait()

        tok, out_b = tok_bufs[cur], out_bufs[cur]

        @plsc.parallel_loop(0, CHUNK, unroll=4)
        def _add(s):
            for gi in range(GROUPS):
                dst = pl.ds((s * GROUPS + gi) * LANES, LANES)
                src = pl.ds(gi * LANES, LANES)
                plsc.addupdate(out_b.at[dst], tok[s, src])

        out_dma[k] = pltpu.async_copy(
            out_b, out_hbm.at[pl.ds((base + k * CHUNK) * DIM, CHUNK * DIM)],
            osems[cur])

    for k in sorted(out_dma):
        out_dma.pop(k).wait()


def kernel(inputs, token_table, pos_table):
    idx = inputs.reshape(-1).astype(jnp.int32)
    table_p = jnp.pad(token_table, ((0, 0), (0, DIMP - DIM)))
    pos2 = pos_table.reshape(-1)
    flat = _emb_kernel(idx, table_p, pos2)
    return flat.reshape(BATCH, SEQ, DIM)


# speedup vs baseline: 1.4814x; 1.4814x over previous
import functools

import jax
import jax.numpy as jnp
from jax import lax
from jax.experimental import pallas as pl
from jax.experimental.pallas import tpu as pltpu
from jax.experimental.pallas import tpu_sc as plsc

VOCAB = 100000
SEQ = 200
DIM = 64
BATCH = 1024
LANES = 16

NUM_CORES = 2
NUM_SUBCORES = 16
NW = NUM_CORES * NUM_SUBCORES
TOK_PER_W = BATCH * SEQ // NW
CHUNK_ROWS = 2
CHUNK = CHUNK_ROWS * SEQ
NCHUNK = TOK_PER_W // CHUNK
NBUF = 3
GROUPS = DIM // LANES

_mesh = plsc.VectorSubcoreMesh(
    core_axis_name="c", subcore_axis_name="s",
    num_cores=NUM_CORES, num_subcores=NUM_SUBCORES)


@functools.partial(
    pl.kernel,
    out_type=jax.ShapeDtypeStruct((BATCH * SEQ, DIM), jnp.float32),
    mesh=_mesh,
    scratch_types=[
        pltpu.VMEM((CHUNK, DIM), jnp.float32),
        pltpu.VMEM((TOK_PER_W,), jnp.int32),
        [pltpu.VMEM((CHUNK, DIM), jnp.float32) for _ in range(NBUF)],
        [pltpu.SemaphoreType.DMA for _ in range(NBUF)],
        [pltpu.SemaphoreType.DMA for _ in range(NBUF)],
    ],
    compiler_params=pltpu.CompilerParams(use_tc_tiling_on_sc=False),
)
def _emb_kernel(idx_hbm, table_hbm, pos_hbm, out_hbm,
                pos_v, idx_v, tok_bufs, gsems, osems):
    wid = lax.axis_index("s") * NUM_CORES + lax.axis_index("c")
    base = wid * TOK_PER_W

    pltpu.sync_copy(idx_hbm.at[pl.ds(base, TOK_PER_W)], idx_v)
    for r in range(CHUNK_ROWS):
        pltpu.sync_copy(pos_hbm, pos_v.at[pl.ds(r * SEQ, SEQ)])

    def start_gather(k, b):
        return pltpu.async_copy(
            table_hbm.at[idx_v.at[pl.ds(k * CHUNK, CHUNK)]], tok_bufs[b],
            gsems[b])

    gather = {0: start_gather(0, 0)}
    out_dma = {}
    for k in range(NCHUNK):
        cur = k % NBUF
        gather.pop(k).wait()
        if k + 1 < NCHUNK:
            nxt = (k + 1) % NBUF
            if k + 1 >= NBUF:
                out_dma.pop(k + 1 - NBUF).wait()
            gather[k + 1] = start_gather(k + 1, nxt)

        tok = tok_bufs[cur]

        @plsc.parallel_loop(0, CHUNK, unroll=8)
        def _add(s):
            for g in range(GROUPS):
                sl = pl.ds(g * LANES, LANES)
                plsc.addupdate(tok.at[s, sl], pos_v[s, sl])

        out_dma[k] = pltpu.async_copy(
            tok, out_hbm.at[pl.ds(base + k * CHUNK, CHUNK)], osems[cur])

    for k in sorted(out_dma):
        out_dma.pop(k).wait()


def kernel(inputs, token_table, pos_table):
    flat = _emb_kernel(inputs.reshape(-1).astype(jnp.int32),
                       token_table, pos_table)
    return flat.reshape(BATCH, SEQ, DIM)
